# baseline (speedup 1.0000x reference)
import functools

import jax
import jax.numpy as jnp
from jax import lax
from jax.experimental import pallas as pl
from jax.experimental.pallas import tpu as pltpu
from jax.experimental.pallas import tpu_sc as plsc


def _positional_table(seq_len, dim):
    even_i = jnp.arange(0, dim, 2).astype(jnp.float32)
    denominator = jnp.power(10000.0, even_i / dim)
    position = jnp.arange(seq_len, dtype=jnp.float32).reshape(seq_len, 1)
    even_pe = jnp.sin(position / denominator)
    odd_pe = jnp.cos(position / denominator)
    return jnp.stack([even_pe, odd_pe], axis=2).reshape(seq_len, dim)


def kernel(x, embedding):
    batch, seq = x.shape
    vocab, dim = embedding.shape
    n = batch * seq

    info = plsc.get_sparse_core_info()
    num_workers = info.num_cores * info.num_subcores

    rows_per_chunk = 2
    chunk = rows_per_chunk * seq
    gather = 100
    ng = chunk // gather
    per_worker = n // num_workers
    chunks_per_worker = per_worker // chunk

    xf = x.reshape(n // gather, gather).astype(jnp.int32)
    pe2 = jnp.tile(_positional_table(seq, dim), (rows_per_chunk, 1))

    mesh = plsc.VectorSubcoreMesh(core_axis_name="c", subcore_axis_name="s")

    @functools.partial(
        pl.kernel,
        mesh=mesh,
        out_type=jax.ShapeDtypeStruct((n, dim), jnp.float32),
        compiler_params=pltpu.CompilerParams(use_tc_tiling_on_sc=False),
        scratch_types=[
            pltpu.VMEM((ng, gather), jnp.int32),
            pltpu.VMEM((chunk, dim), jnp.float32),
            pltpu.VMEM((chunk, dim), jnp.float32),
            pltpu.SemaphoreType.DMA,
        ],
    )
    def sc_kernel(xf_hbm, emb_hbm, pe_hbm, out_hbm, idx_v, rows_v, pe_v, sem):
        cid = lax.axis_index("c")
        sid = lax.axis_index("s")
        wid = sid * info.num_cores + cid
        pltpu.sync_copy(pe_hbm, pe_v)
        chunk0 = wid * chunks_per_worker

        def chunk_body(g, carry):
            ch = chunk0 + g
            pltpu.sync_copy(xf_hbm.at[pl.ds(ch * ng, ng)], idx_v)
            copies = [
                pltpu.async_copy(
                    emb_hbm.at[idx_v.at[i]],
                    rows_v.at[pl.ds(i * gather, gather)],
                    sem,
                )
                for i in range(ng)
            ]
            for cp in copies:
                cp.wait()

            def add_body(r, c2):
                for dg in range(dim // 16):
                    sl = pl.ds(dg * 16, 16)
                    rows_v[r, sl] = rows_v[r, sl] + pe_v[r, sl]
                return c2

            lax.fori_loop(0, chunk, add_body, 0)
            pltpu.sync_copy(rows_v, out_hbm.at[pl.ds(ch * chunk, chunk)])
            return carry

        lax.fori_loop(0, chunks_per_worker, chunk_body, 0)

    out = sc_kernel(xf, embedding, pe2)
    return out.reshape(batch, seq, dim)


# speedup vs baseline: 1.0989x; 1.0989x over previous
import functools

import jax
import jax.numpy as jnp
from jax import lax
from jax.experimental import pallas as pl
from jax.experimental.pallas import tpu as pltpu
from jax.experimental.pallas import tpu_sc as plsc


def _positional_table(seq_len, dim):
    even_i = jnp.arange(0, dim, 2).astype(jnp.float32)
    denominator = jnp.power(10000.0, even_i / dim)
    position = jnp.arange(seq_len, dtype=jnp.float32).reshape(seq_len, 1)
    even_pe = jnp.sin(position / denominator)
    odd_pe = jnp.cos(position / denominator)
    return jnp.stack([even_pe, odd_pe], axis=2).reshape(seq_len, dim)


def kernel(x, embedding):
    batch, seq = x.shape
    vocab, dim = embedding.shape
    n = batch * seq

    info = plsc.get_sparse_core_info()
    num_workers = info.num_cores * info.num_subcores

    rows_per_chunk = 2
    chunk = rows_per_chunk * seq
    gather = 100
    ng = chunk // gather
    per_worker = n // num_workers
    n_chunks = per_worker // chunk
    n_pairs = n_chunks // 2

    xf = x.reshape(n // gather, gather).astype(jnp.int32)
    pe2 = jnp.tile(_positional_table(seq, dim), (rows_per_chunk, 1))

    mesh = plsc.VectorSubcoreMesh(core_axis_name="c", subcore_axis_name="s")

    @functools.partial(
        pl.kernel,
        mesh=mesh,
        out_type=jax.ShapeDtypeStruct((n, dim), jnp.float32),
        compiler_params=pltpu.CompilerParams(use_tc_tiling_on_sc=False),
        scratch_types=[
            pltpu.VMEM((ng, gather), jnp.int32),
            pltpu.VMEM((ng, gather), jnp.int32),
            pltpu.VMEM((chunk, dim), jnp.float32),
            pltpu.VMEM((chunk, dim), jnp.float32),
            pltpu.VMEM((chunk, dim), jnp.float32),
            pltpu.SemaphoreType.DMA,
            pltpu.SemaphoreType.DMA,
            pltpu.SemaphoreType.DMA,
            pltpu.SemaphoreType.DMA,
        ],
    )
    def sc_kernel(xf_hbm, emb_hbm, pe_hbm, out_hbm,
                  idx0, idx1, rows0, rows1, pe_v,
                  semg0, semg1, sems0, sems1):
        cid = lax.axis_index("c")
        sid = lax.axis_index("s")
        wid = sid * info.num_cores + cid
        pltpu.sync_copy(pe_hbm, pe_v)
        chunk0 = wid * n_chunks

        def load_idx(ch, idx_v):
            pltpu.sync_copy(xf_hbm.at[pl.ds(ch * ng, ng)], idx_v)

        def fire_gathers(idx_v, rows_v, semg):
            for i in range(ng):
                pltpu.make_async_copy(
                    emb_hbm.at[idx_v.at[i]],
                    rows_v.at[pl.ds(i * gather, gather)],
                    semg,
                ).start()

        def wait_gathers(idx_v, rows_v, semg):
            for i in range(ng):
                pltpu.make_async_copy(
                    emb_hbm.at[idx_v.at[i]],
                    rows_v.at[pl.ds(i * gather, gather)],
                    semg,
                ).wait()

        def add_pe(rows_v):
            @plsc.parallel_loop(0, chunk, unroll=4)
            def _(r):
                for dg in range(dim // 16):
                    sl = pl.ds(dg * 16, 16)
                    rows_v[r, sl] = rows_v[r, sl] + pe_v[r, sl]

        def fire_store(ch, rows_v, sems):
            pltpu.make_async_copy(
                rows_v, out_hbm.at[pl.ds(ch * chunk, chunk)], sems
            ).start()

        def wait_store(ch, rows_v, sems):
            pltpu.make_async_copy(
                rows_v, out_hbm.at[pl.ds(ch * chunk, chunk)], sems
            ).wait()

        load_idx(chunk0, idx0)
        fire_gathers(idx0, rows0, semg0)
        load_idx(chunk0 + 1, idx1)
        fire_gathers(idx1, rows1, semg1)

        def pair_body(p, carry):
            ch_a = chunk0 + 2 * p
            ch_b = ch_a + 1
            wait_gathers(idx0, rows0, semg0)
            add_pe(rows0)
            fire_store(ch_a, rows0, sems0)
            wait_gathers(idx1, rows1, semg1)
            add_pe(rows1)
            fire_store(ch_b, rows1, sems1)
            wait_store(ch_a, rows0, sems0)

            @pl.when(p < n_pairs - 1)
            def _():
                load_idx(ch_a + 2, idx0)
                fire_gathers(idx0, rows0, semg0)

            wait_store(ch_b, rows1, sems1)

            @pl.when(p < n_pairs - 1)
            def _():
                load_idx(ch_b + 2, idx1)
                fire_gathers(idx1, rows1, semg1)

            return carry

        lax.fori_loop(0, n_pairs, pair_body, 0)

    out = sc_kernel(xf, embedding, pe2)
    return out.reshape(batch, seq, dim)


# speedup vs baseline: 1.1029x; 1.0036x over previous
import functools

import jax
import jax.numpy as jnp
from jax import lax
from jax.experimental import pallas as pl
from jax.experimental.pallas import tpu as pltpu
from jax.experimental.pallas import tpu_sc as plsc


def _positional_table(seq_len, dim):
    even_i = jnp.arange(0, dim, 2).astype(jnp.float32)
    denominator = jnp.power(10000.0, even_i / dim)
    position = jnp.arange(seq_len, dtype=jnp.float32).reshape(seq_len, 1)
    even_pe = jnp.sin(position / denominator)
    odd_pe = jnp.cos(position / denominator)
    return jnp.stack([even_pe, odd_pe], axis=2).reshape(seq_len, dim)


def kernel(x, embedding):
    batch, seq = x.shape
    vocab, dim = embedding.shape
    n = batch * seq

    info = plsc.get_sparse_core_info()
    num_workers = info.num_cores * info.num_subcores

    rows_per_chunk = 2
    chunk = rows_per_chunk * seq
    splits = []
    off = 0
    while off < chunk:
        size = min(128, chunk - off)
        splits.append((off, size))
        off += size
    per_worker = n // num_workers
    n_chunks = per_worker // chunk
    n_pairs = n_chunks // 2

    xf = x.reshape(n).astype(jnp.int32)
    pe2 = jnp.tile(_positional_table(seq, dim), (rows_per_chunk, 1)).reshape(
        chunk * dim
    )

    mesh = plsc.VectorSubcoreMesh(core_axis_name="c", subcore_axis_name="s")

    @functools.partial(
        pl.kernel,
        mesh=mesh,
        out_type=jax.ShapeDtypeStruct((n, dim), jnp.float32),
        compiler_params=pltpu.CompilerParams(use_tc_tiling_on_sc=False),
        scratch_types=[
            pltpu.VMEM((chunk,), jnp.int32),
            pltpu.VMEM((chunk,), jnp.int32),
            pltpu.VMEM((chunk, dim), jnp.float32),
            pltpu.VMEM((chunk, dim), jnp.float32),
            pltpu.VMEM((chunk * dim,), jnp.float32),
            pltpu.SemaphoreType.DMA,
            pltpu.SemaphoreType.DMA,
            pltpu.SemaphoreType.DMA,
            pltpu.SemaphoreType.DMA,
        ],
    )
    def sc_kernel(xf_hbm, emb_hbm, pe_hbm, out_hbm,
                  idx0, idx1, rows0, rows1, pe_v,
                  semg0, semg1, sems0, sems1):
        cid = lax.axis_index("c")
        sid = lax.axis_index("s")
        wid = sid * info.num_cores + cid
        pltpu.sync_copy(pe_hbm, pe_v)
        chunk0 = wid * n_chunks

        def load_idx(ch, idx_v):
            pltpu.sync_copy(xf_hbm.at[pl.ds(ch * chunk, chunk)], idx_v)

        def fire_gathers(idx_v, rows_v, semg):
            for off, size in splits:
                pltpu.make_async_copy(
                    emb_hbm.at[idx_v.at[pl.ds(off, size)]],
                    rows_v.at[pl.ds(off, size)],
                    semg,
                ).start()

        def wait_gathers(idx_v, rows_v, semg):
            for off, size in splits:
                pltpu.make_async_copy(
                    emb_hbm.at[idx_v.at[pl.ds(off, size)]],
                    rows_v.at[pl.ds(off, size)],
                    semg,
                ).wait()

        def add_pe(rows_v):
            @plsc.parallel_loop(0, chunk, unroll=4)
            def _(r):
                for dg in range(dim // 16):
                    sl = pl.ds(dg * 16, 16)
                    rows_v[r, sl] = rows_v[r, sl] + pe_v[pl.ds(r * dim + dg * 16, 16)]

        def fire_store(ch, rows_v, sems):
            pltpu.make_async_copy(
                rows_v, out_hbm.at[pl.ds(ch * chunk, chunk)], sems
            ).start()

        def wait_store(ch, rows_v, sems):
            pltpu.make_async_copy(
                rows_v, out_hbm.at[pl.ds(ch * chunk, chunk)], sems
            ).wait()

        load_idx(chunk0, idx0)
        fire_gathers(idx0, rows0, semg0)
        load_idx(chunk0 + 1, idx1)
        fire_gathers(idx1, rows1, semg1)

        def pair_body(p, carry):
            ch_a = chunk0 + 2 * p
            ch_b = ch_a + 1
            wait_gathers(idx0, rows0, semg0)
            add_pe(rows0)
            fire_store(ch_a, rows0, sems0)
            wait_gathers(idx1, rows1, semg1)
            add_pe(rows1)
            fire_store(ch_b, rows1, sems1)
            wait_store(ch_a, rows0, sems0)

            @pl.when(p < n_pairs - 1)
            def _():
                load_idx(ch_a + 2, idx0)
                fire_gathers(idx0, rows0, semg0)

            wait_store(ch_b, rows1, sems1)

            @pl.when(p < n_pairs - 1)
            def _():
                load_idx(ch_b + 2, idx1)
                fire_gathers(idx1, rows1, semg1)

            return carry

        lax.fori_loop(0, n_pairs, pair_body, 0)

    out = sc_kernel(xf, embedding, pe2)
    return out.reshape(batch, seq, dim)


# speedup vs baseline: 1.1049x; 1.0018x over previous
import functools

import jax
import jax.numpy as jnp
from jax import lax
from jax.experimental import pallas as pl
from jax.experimental.pallas import tpu as pltpu
from jax.experimental.pallas import tpu_sc as plsc


def _positional_table(seq_len, dim):
    even_i = jnp.arange(0, dim, 2).astype(jnp.float32)
    denominator = jnp.power(10000.0, even_i / dim)
    position = jnp.arange(seq_len, dtype=jnp.float32).reshape(seq_len, 1)
    even_pe = jnp.sin(position / denominator)
    odd_pe = jnp.cos(position / denominator)
    return jnp.stack([even_pe, odd_pe], axis=2).reshape(seq_len, dim)


def kernel(x, embedding):
    batch, seq = x.shape
    vocab, dim = embedding.shape

    info = plsc.get_sparse_core_info()
    num_workers = info.num_cores * info.num_subcores

    rows_per_chunk = 2
    chunk = rows_per_chunk * seq
    per_worker = batch // num_workers
    n_chunks = per_worker // rows_per_chunk
    n_pairs = n_chunks // 2

    splits = []
    off = 0
    while off < seq:
        size = min(128, seq - off)
        splits.append((off, size))
        off += size

    xi = x.astype(jnp.int32)
    pe1 = _positional_table(seq, dim).reshape(seq * dim)

    mesh = plsc.VectorSubcoreMesh(core_axis_name="c", subcore_axis_name="s")

    @functools.partial(
        pl.kernel,
        mesh=mesh,
        out_type=jax.ShapeDtypeStruct((batch, seq, dim), jnp.float32),
        compiler_params=pltpu.CompilerParams(use_tc_tiling_on_sc=False),
        scratch_types=[
            pltpu.VMEM((rows_per_chunk, seq), jnp.int32),
            pltpu.VMEM((rows_per_chunk, seq), jnp.int32),
            pltpu.VMEM((rows_per_chunk, seq, dim), jnp.float32),
            pltpu.VMEM((rows_per_chunk, seq, dim), jnp.float32),
            pltpu.VMEM((seq * dim,), jnp.float32),
            pltpu.SemaphoreType.DMA,
            pltpu.SemaphoreType.DMA,
            pltpu.SemaphoreType.DMA,
            pltpu.SemaphoreType.DMA,
        ],
    )
    def sc_kernel(xi_hbm, emb_hbm, pe_hbm, out_hbm,
                  idx0, idx1, rows0, rows1, pe_v,
                  semg0, semg1, sems0, sems1):
        cid = lax.axis_index("c")
        sid = lax.axis_index("s")
        wid = sid * info.num_cores + cid
        pltpu.sync_copy(pe_hbm, pe_v)
        row_base = wid * per_worker

        def load_idx(ch, idx_v):
            pltpu.sync_copy(
                xi_hbm.at[pl.ds(row_base + ch * rows_per_chunk, rows_per_chunk)],
                idx_v,
            )

        def gather_copies(idx_v, rows_v, semg):
            return [
                pltpu.make_async_copy(
                    emb_hbm.at[idx_v.at[i, pl.ds(off, size)]],
                    rows_v.at[i, pl.ds(off, size)],
                    semg,
                )
                for i in range(rows_per_chunk)
                for off, size in splits
            ]

        def add_pe(rows_v):
            for i in range(rows_per_chunk):
                @plsc.parallel_loop(0, seq, unroll=4)
                def _(r):
                    for dg in range(dim // 16):
                        sl = pl.ds(dg * 16, 16)
                        rows_v[i, r, sl] = rows_v[i, r, sl] + pe_v[
                            pl.ds(r * dim + dg * 16, 16)
                        ]

        def store_copy(ch, rows_v, sems):
            return pltpu.make_async_copy(
                rows_v,
                out_hbm.at[pl.ds(row_base + ch * rows_per_chunk, rows_per_chunk)],
                sems,
            )

        load_idx(0, idx0)
        for cp in gather_copies(idx0, rows0, semg0):
            cp.start()
        load_idx(1, idx1)
        for cp in gather_copies(idx1, rows1, semg1):
            cp.start()

        def pair_body(p, carry):
            ch_a = 2 * p
            ch_b = ch_a + 1
            for cp in gather_copies(idx0, rows0, semg0):
                cp.wait()
            add_pe(rows0)
            store_copy(ch_a, rows0, sems0).start()
            for cp in gather_copies(idx1, rows1, semg1):
                cp.wait()
            add_pe(rows1)
            store_copy(ch_b, rows1, sems1).start()
            store_copy(ch_a, rows0, sems0).wait()

            @pl.when(p < n_pairs - 1)
            def _():
                load_idx(ch_a + 2, idx0)
                for cp in gather_copies(idx0, rows0, semg0):
                    cp.start()

            store_copy(ch_b, rows1, sems1).wait()

            @pl.when(p < n_pairs - 1)
            def _():
                load_idx(ch_b + 2, idx1)
                for cp in gather_copies(idx1, rows1, semg1):
                    cp.start()

            return carry

        lax.fori_loop(0, n_pairs, pair_body, 0)

    return sc_kernel(xi, embedding, pe1)
